# baseline (speedup 1.0000x reference)
import functools

import jax
import jax.numpy as jnp
from jax import lax
from jax.experimental import pallas as pl
from jax.experimental.pallas import tpu as pltpu
from jax.experimental.pallas import tpu_sc as plsc

_D = 64
_DP = 128
_IV = 128
_K = 2
_W = _IV * _K
_T = 200
_NW = 32
_NB = 16384


def _build():
    mesh = plsc.VectorSubcoreMesh(core_axis_name="c", subcore_axis_name="s")
    n_t = _T

    @functools.partial(
        pl.kernel,
        mesh=mesh,
        out_type=jax.ShapeDtypeStruct((_T, _D, _NB), jnp.float32),
        compiler_params=pltpu.CompilerParams(needs_layout_passes=False),
        scratch_types=[
            pltpu.VMEM((_W,), jnp.int32),
            pltpu.VMEM((_W,), jnp.int32),
            pltpu.VMEM((_W, _DP), jnp.float32),
            pltpu.VMEM((_W, _DP), jnp.float32),
            pltpu.VMEM((_D, _W), jnp.float32),
            pltpu.VMEM((_D, _W), jnp.float32),
            pltpu.SemaphoreType.DMA,
            pltpu.SemaphoreType.DMA,
            pltpu.SemaphoreType.DMA,
        ],
    )
    def gather_kernel(table_hbm, xt_hbm, out_hbm, idx0, idx1, rows0, rows1,
                      tr0, tr1, isem, gsem, osem):
        wid = lax.axis_index("s") * 2 + lax.axis_index("c")
        b0 = wid * (_NB // _NW)

        def idx_copy(t, h, idx_s):
            return pltpu.make_async_copy(
                xt_hbm.at[t, pl.ds(b0 + h * _W, _W)], idx_s, isem)

        def gather_copy(idx_s, rows_s, j):
            return pltpu.make_async_copy(
                table_hbm.at[idx_s.at[pl.ds(j * _IV, _IV)]],
                rows_s.at[pl.ds(j * _IV, _IV)], gsem)

        def out_copy(t, h, tr_s):
            return pltpu.make_async_copy(
                tr_s, out_hbm.at[t, :, pl.ds(b0 + h * _W, _W)], osem)

        def transpose(rows_s, tr_s):
            lanes = lax.iota(jnp.int32, 16)
            perms = [(lanes + k) & 15 for k in range(16)]

            def tbody(ib, carry):
                row_idx = ib * 16 + lanes
                for dg in range(0, _D // 16, 2):
                    dcols = [dg * 16 + p for p in perms] + \
                            [(dg + 1) * 16 + p for p in perms]
                    vs = [plsc.load_gather(rows_s, [row_idx, dc])
                          for dc in dcols]
                    for dc, v in zip(dcols, vs):
                        plsc.store_scatter(tr_s, [dc, row_idx], v)
                return carry
            lax.fori_loop(0, _W // 16, tbody, 0)

        def unit(g, h, idx_s, rows_s, tr_s, o_idx, o_rows, o_tr):
            for j in range(_K):
                gather_copy(idx_s, rows_s, j).wait()

            if h == 0:
                @pl.when(g > 0)
                def _():
                    out_copy(g - 1, 1, o_tr).wait()
            else:
                out_copy(g, 0, o_tr).wait()

            if h == 0:
                idx_copy(g, 1, o_idx).wait()
                for j in range(_K):
                    gather_copy(o_idx, o_rows, j).start()
            else:
                @pl.when(g + 1 < n_t)
                def _():
                    idx_copy(g + 1, 0, o_idx).wait()
                    for j in range(_K):
                        gather_copy(o_idx, o_rows, j).start()

            @pl.when(g + 1 < n_t)
            def _():
                idx_copy(g + 1, h, idx_s).start()

            transpose(rows_s, tr_s)
            out_copy(g, h, tr_s).start()

        idx_copy(0, 0, idx0).start()
        idx_copy(0, 1, idx1).start()
        idx_copy(0, 0, idx0).wait()
        for j in range(_K):
            gather_copy(idx0, rows0, j).start()

        def body(g, carry):
            unit(g, 0, idx0, rows0, tr0, idx1, rows1, tr1)
            unit(g, 1, idx1, rows1, tr1, idx0, rows0, tr0)
            return carry

        lax.fori_loop(0, n_t, body, 0)
        out_copy(n_t - 1, 1, tr1).wait()

    return gather_kernel


@jax.jit
def kernel(x, table):
    table_p = jnp.pad(table, ((0, 0), (0, _DP - _D)))
    out_p = _build()(table_p, x.T)
    return jnp.transpose(out_p, (2, 0, 1))


# speedup vs baseline: 1.2370x; 1.2370x over previous
import functools

import jax
import jax.numpy as jnp
from jax import lax
from jax.experimental import pallas as pl
from jax.experimental.pallas import tpu as pltpu
from jax.experimental.pallas import tpu_sc as plsc

_D = 64
_DP = 128
_IV = 128
_K = 2
_W = _IV * _K
_T = 200
_NW = 32
_NB = 16384


def _build():
    mesh = plsc.VectorSubcoreMesh(core_axis_name="c", subcore_axis_name="s")
    n_t = _T

    @functools.partial(
        pl.kernel,
        mesh=mesh,
        out_type=jax.ShapeDtypeStruct((_T, _D, _NB), jnp.float32),
        compiler_params=pltpu.CompilerParams(needs_layout_passes=False),
        scratch_types=[
            pltpu.VMEM((_W,), jnp.int32),
            pltpu.VMEM((_W,), jnp.int32),
            pltpu.VMEM((_W, _DP), jnp.float32),
            pltpu.VMEM((_W, _DP), jnp.float32),
            pltpu.VMEM((_D, _W), jnp.float32),
            pltpu.VMEM((_D, _W), jnp.float32),
            pltpu.SemaphoreType.DMA,
            pltpu.SemaphoreType.DMA,
            pltpu.SemaphoreType.DMA,
        ],
    )
    def gather_kernel(table_hbm, xt_hbm, out_hbm, idx0, idx1, rows0, rows1,
                      tr0, tr1, isem, gsem, osem):
        wid = lax.axis_index("s") * 2 + lax.axis_index("c")
        b0 = wid * (_NB // _NW)

        def idx_copy(t, h, idx_s):
            return pltpu.make_async_copy(
                xt_hbm.at[t, pl.ds(b0 + h * _W, _W)], idx_s, isem)

        def gather_copy(idx_s, rows_s, j):
            return pltpu.make_async_copy(
                table_hbm.at[idx_s.at[pl.ds(j * _IV, _IV)]],
                rows_s.at[pl.ds(j * _IV, _IV)], gsem)

        def out_copy(t, h, tr_s):
            return pltpu.make_async_copy(
                tr_s, out_hbm.at[t, :, pl.ds(b0 + h * _W, _W)], osem)

        def transpose(rows_s, tr_s):
            lanes = lax.iota(jnp.int32, 16)
            perms = [(lanes + k) & 15 for k in range(16)]

            def tbody(ib, carry):
                row_idx = ib * 16 + lanes
                for dg in range(_D // 16):
                    dcols = [dg * 16 + p for p in perms]
                    vs = [plsc.load_gather(rows_s, [row_idx, dc])
                          for dc in dcols]
                    for dc, v in zip(dcols, vs):
                        plsc.store_scatter(tr_s, [dc, row_idx], v)
                return carry
            lax.fori_loop(0, _W // 16, tbody, 0)

        def unit(g, h, idx_s, rows_s, tr_s, o_idx, o_rows, o_tr):
            for j in range(_K):
                gather_copy(idx_s, rows_s, j).wait()

            if h == 0:
                @pl.when(g > 0)
                def _():
                    out_copy(g - 1, 1, o_tr).wait()
            else:
                out_copy(g, 0, o_tr).wait()

            if h == 0:
                idx_copy(g, 1, o_idx).wait()
                for j in range(_K):
                    gather_copy(o_idx, o_rows, j).start()
            else:
                @pl.when(g + 1 < n_t)
                def _():
                    idx_copy(g + 1, 0, o_idx).wait()
                    for j in range(_K):
                        gather_copy(o_idx, o_rows, j).start()

            @pl.when(g + 1 < n_t)
            def _():
                idx_copy(g + 1, h, idx_s).start()

            transpose(rows_s, tr_s)
            out_copy(g, h, tr_s).start()

        idx_copy(0, 0, idx0).start()
        idx_copy(0, 1, idx1).start()
        idx_copy(0, 0, idx0).wait()
        for j in range(_K):
            gather_copy(idx0, rows0, j).start()

        def body(g, carry):
            unit(g, 0, idx0, rows0, tr0, idx1, rows1, tr1)
            unit(g, 1, idx1, rows1, tr1, idx0, rows0, tr0)
            return carry

        lax.fori_loop(0, n_t, body, 0)
        out_copy(n_t - 1, 1, tr1).wait()

    return gather_kernel


_NREP = 8


@jax.jit
def kernel(x, table):
    table_p = jnp.pad(table, ((0, 0), (0, _DP - _D)))
    table_r = jnp.tile(table_p, (_NREP, 1))
    rep = (jnp.arange(_NB, dtype=jnp.int32) // (_NB // _NW)) % _NREP
    x_adj = x + rep[:, None] * (table.shape[0])
    out_p = _build()(table_r, x_adj.T)
    return jnp.transpose(out_p, (2, 0, 1))
